# Initial kernel scaffold; baseline (speedup 1.0000x reference)
import jax
import jax.numpy as jnp
from jax.experimental import pallas as pl


def kernel(x, edge_index, W, att_src, att_dst, bias, bn_gamma, bn_beta, fc_W, fc_b):
    raise NotImplementedError("write your pallas kernel here")


# speedup vs baseline: 22.4889x; 22.4889x over previous
import functools
import math

import jax
import jax.numpy as jnp
from jax import lax
from jax.experimental import pallas as pl
from jax.experimental.pallas import tpu as pltpu
from jax.experimental.pallas import tpu_sc as plsc

N0 = 10000
E0 = 320000
D = 128
H = 128
O = 128

LANES = 16
NP = 10240
C = 128
NW = 32
NCH = -(-E0 // (NW * C))
EP = NCH * NW * C
RPT = NP // LANES
BR = 256


def _dense_body(x_ref, w_ref, att_ref, h_ref, ab_ref):
    h = jnp.dot(x_ref[...], w_ref[...], preferred_element_type=jnp.float32)
    h_ref[...] = h
    ab_ref[...] = jnp.dot(h, att_ref[...], preferred_element_type=jnp.float32)


def _dense1(x_p, W, att_mat):
    return pl.pallas_call(
        _dense_body,
        grid=(NP // BR,),
        in_specs=[
            pl.BlockSpec((BR, D), lambda i: (i, 0)),
            pl.BlockSpec((D, H), lambda i: (0, 0)),
            pl.BlockSpec((H, 2), lambda i: (0, 0)),
        ],
        out_specs=[
            pl.BlockSpec((BR, H), lambda i: (i, 0)),
            pl.BlockSpec((BR, 2), lambda i: (i, 0)),
        ],
        out_shape=[
            jax.ShapeDtypeStruct((NP, H), jnp.float32),
            jax.ShapeDtypeStruct((NP, 2), jnp.float32),
        ],
    )(x_p, W, att_mat)


def _sc_body(src_hbm, dst_hbm, a_hbm, b_hbm, h_hbm, acc_out, den_out,
             a_v, b_v, si, di, wv, rows, accS, denS, sem):
    cid = lax.axis_index("c")
    sid = lax.axis_index("s")
    wid = sid * 2 + cid

    pltpu.sync_copy(a_hbm, a_v)
    pltpu.sync_copy(b_hbm, b_v)

    zeros16 = jnp.zeros((LANES,), jnp.float32)

    def _zrow(j, carry):
        for k in range(H // LANES):
            rows[j, pl.ds(k * LANES, LANES)] = zeros16
        return carry

    lax.fori_loop(0, C, _zrow, 0)
    for k in range(C // LANES):
        wv[pl.ds(k * LANES, LANES)] = zeros16
    r0 = sid * RPT
    for q in range(RPT // C):
        pltpu.sync_copy(rows, accS.at[pl.ds(r0 + q * C, C)])
        pltpu.sync_copy(wv, denS.at[pl.ds(r0 + q * C, C)])
    plsc.subcore_barrier()

    def _chunk(i, carry):
        base = pl.multiple_of((wid * NCH + i) * C, C)
        pltpu.sync_copy(src_hbm.at[pl.ds(base, C)], si)
        pltpu.sync_copy(dst_hbm.at[pl.ds(base, C)], di)
        for j in range(C // LANES):
            s16 = si[pl.ds(j * LANES, LANES)]
            d16 = di[pl.ds(j * LANES, LANES)]
            e = plsc.load_gather(a_v, [s16]) + plsc.load_gather(b_v, [d16])
            e = jnp.where(e < 0.0, e * 0.2, e)
            wv[pl.ds(j * LANES, LANES)] = jnp.exp(e)
        pltpu.async_copy(h_hbm.at[si], rows, sem).wait()

        def _scale(g, c2):
            w16 = wv[pl.ds(g * LANES, LANES)]
            for t in range(LANES):
                w = w16[t]
                for k in range(H // LANES):
                    sl = pl.ds(k * LANES, LANES)
                    rows[g * LANES + t, sl] = rows[g * LANES + t, sl] * w
            return c2

        lax.fori_loop(0, C // LANES, _scale, 0)
        pltpu.sync_copy(rows, accS.at[di], add=True)
        pltpu.sync_copy(wv, denS.at[di], add=True)
        return carry

    lax.fori_loop(0, NCH, _chunk, 0)
    plsc.subcore_barrier()

    pltpu.sync_copy(accS.at[pl.ds(r0, RPT)], acc_out.at[cid, pl.ds(r0, RPT)])
    pltpu.sync_copy(denS.at[pl.ds(r0, RPT)], den_out.at[cid, pl.ds(r0, RPT)])


def _sc_aggregate(src, dst, a, b, h):
    mesh = plsc.VectorSubcoreMesh(core_axis_name="c", subcore_axis_name="s")
    kern = functools.partial(
        pl.kernel,
        mesh=mesh,
        compiler_params=pltpu.CompilerParams(needs_layout_passes=False),
        out_type=[
            jax.ShapeDtypeStruct((2, NP, H), jnp.float32),
            jax.ShapeDtypeStruct((2, NP), jnp.float32),
        ],
        scratch_types=[
            pltpu.VMEM((NP,), jnp.float32),
            pltpu.VMEM((NP,), jnp.float32),
            pltpu.VMEM((C,), jnp.int32),
            pltpu.VMEM((C,), jnp.int32),
            pltpu.VMEM((C,), jnp.float32),
            pltpu.VMEM((C, H), jnp.float32),
            pltpu.VMEM_SHARED((NP, H), jnp.float32),
            pltpu.VMEM_SHARED((NP,), jnp.float32),
            pltpu.SemaphoreType.DMA,
        ],
    )(_sc_body)
    return kern(src, dst, a, b, h)


_BN_SCALE = 1.0 / math.sqrt(1.0 + 1e-5)


def _epi_body(acc_ref, den_ref, h_ref, ab_ref, bias_ref, gam_ref, bet_ref,
              fcw_ref, fcb_ref, o_ref):
    i = pl.program_id(0)
    den2 = den_ref[:, pl.ds(i * BR, BR)]
    ab = ab_ref[...]
    e = ab[:, 0] + ab[:, 1]
    e = jnp.where(e < 0.0, e * 0.2, e)
    ws = jnp.exp(e)
    acc = acc_ref[0] + acc_ref[1] + ws[:, None] * h_ref[...]
    den = den2[0] + den2[1] + ws
    node = acc / (den + 1e-16)[:, None]
    node = node + bias_ref[...]
    node = node * _BN_SCALE * gam_ref[...] + bet_ref[...]
    node = jnp.maximum(node, 0.0)
    o_ref[...] = jnp.dot(node, fcw_ref[...],
                         preferred_element_type=jnp.float32) + fcb_ref[...]


def _epilogue(accp, denp, h, ab, bias, gam, bet, fc_W, fc_b):
    return pl.pallas_call(
        _epi_body,
        grid=(NP // BR,),
        in_specs=[
            pl.BlockSpec((2, BR, H), lambda i: (0, i, 0)),
            pl.BlockSpec((2, NP), lambda i: (0, 0)),
            pl.BlockSpec((BR, H), lambda i: (i, 0)),
            pl.BlockSpec((BR, 2), lambda i: (i, 0)),
            pl.BlockSpec((1, H), lambda i: (0, 0)),
            pl.BlockSpec((1, H), lambda i: (0, 0)),
            pl.BlockSpec((1, H), lambda i: (0, 0)),
            pl.BlockSpec((H, O), lambda i: (0, 0)),
            pl.BlockSpec((1, O), lambda i: (0, 0)),
        ],
        out_specs=pl.BlockSpec((BR, O), lambda i: (i, 0)),
        out_shape=jax.ShapeDtypeStruct((NP, O), jnp.float32),
    )(accp, denp, h, ab, bias, gam, bet, fc_W, fc_b)


def kernel(x, edge_index, W, att_src, att_dst, bias, bn_gamma, bn_beta,
           fc_W, fc_b):
    x_p = jnp.pad(x, ((0, NP - N0), (0, 0)))
    pad = EP - E0
    src = jnp.concatenate(
        [edge_index[0].astype(jnp.int32), jnp.full((pad,), N0, jnp.int32)])
    dst = jnp.concatenate(
        [edge_index[1].astype(jnp.int32), jnp.full((pad,), N0, jnp.int32)])
    att_mat = jnp.stack([att_src, att_dst], axis=1)

    h, ab = _dense1(x_p, W, att_mat)
    a = ab[:, 0] + 0.0
    b = ab[:, 1] + 0.0
    accp, denp = _sc_aggregate(src, dst, a, b, h)
    out = _epilogue(accp, denp, h, ab, bias.reshape(1, H),
                    bn_gamma.reshape(1, H), bn_beta.reshape(1, H),
                    fc_W, fc_b.reshape(1, O))
    return out[:N0]


# speedup vs baseline: 30.8830x; 1.3733x over previous
import functools
import math

import jax
import jax.numpy as jnp
from jax import lax
from jax.experimental import pallas as pl
from jax.experimental.pallas import tpu as pltpu
from jax.experimental.pallas import tpu_sc as plsc

N0 = 10000
E0 = 320000
D = 128
H = 128
O = 128

LANES = 16
NP = 10240
C = 128
HC = H // 2
NT = 16
NCH = 160
EP = NCH * NT * C
NBUF = 4
RPT = NP // LANES
BR = 256


def _dense_body(x_ref, w_ref, att_ref, h_ref, ab_ref):
    h = jnp.dot(x_ref[...], w_ref[...], preferred_element_type=jnp.float32)
    h_ref[...] = h
    ab_ref[...] = jnp.dot(h, att_ref[...], preferred_element_type=jnp.float32)


def _dense1(x_p, W, att_mat):
    return pl.pallas_call(
        _dense_body,
        grid=(NP // BR,),
        in_specs=[
            pl.BlockSpec((BR, D), lambda i: (i, 0)),
            pl.BlockSpec((D, H), lambda i: (0, 0)),
            pl.BlockSpec((H, 2), lambda i: (0, 0)),
        ],
        out_specs=[
            pl.BlockSpec((BR, H), lambda i: (i, 0)),
            pl.BlockSpec((BR, 2), lambda i: (i, 0)),
        ],
        out_shape=[
            jax.ShapeDtypeStruct((NP, H), jnp.float32),
            jax.ShapeDtypeStruct((NP, 2), jnp.float32),
        ],
    )(x_p, W, att_mat)


def _sc_body(src_hbm, dst_hbm, a_hbm, b_hbm, h2_hbm, acc_out, den_out,
             a_v, b_v, si, di, wv, wz, rows4, accS, denS,
             sg0, sg1, sg2, sg3, ss0, ss1, ss2, ss3):
    cid = lax.axis_index("c")
    sid = lax.axis_index("s")
    semg = [sg0, sg1, sg2, sg3]
    sems = [ss0, ss1, ss2, ss3]

    pltpu.sync_copy(a_hbm, a_v)
    pltpu.sync_copy(b_hbm, b_v)

    zeros16 = jnp.zeros((LANES,), jnp.float32)
    izeros16 = jnp.zeros((LANES,), jnp.int32)

    def _zrow(j, carry):
        for p in range(NBUF):
            for k in range(HC // LANES):
                rows4[p, j, pl.ds(k * LANES, LANES)] = zeros16
        return carry

    lax.fori_loop(0, C, _zrow, 0)
    for k in range(C // LANES):
        wz[pl.ds(k * LANES, LANES)] = zeros16
        for p in range(NBUF):
            di[p, pl.ds(k * LANES, LANES)] = izeros16

    r0 = sid * RPT
    for q in range(RPT // C):
        pltpu.sync_copy(rows4.at[0], accS.at[pl.ds(r0 + q * C, C), :])
        pltpu.sync_copy(wz, denS.at[pl.ds(r0 + q * C, C)])
    plsc.subcore_barrier()

    def _prep(i, p):
        base = pl.multiple_of((sid * NCH + i) * C, C)
        pltpu.sync_copy(src_hbm.at[pl.ds(base, C)], si.at[p])
        pltpu.sync_copy(dst_hbm.at[pl.ds(base, C)], di.at[p])
        for j in range(C // LANES):
            sl = pl.ds(j * LANES, LANES)
            s16 = si[p, sl]
            d16 = di[p, sl]
            e = plsc.load_gather(a_v, [s16]) + plsc.load_gather(b_v, [d16])
            e = jnp.where(e < 0.0, e * 0.2, e)
            wv[p, sl] = jnp.exp(e)
            si[p, sl] = s16 * 2 + cid
        pltpu.async_copy(h2_hbm.at[si.at[p]], rows4.at[p], semg[p])

    for p in range(1, NBUF):
        pltpu.async_copy(rows4.at[p], accS.at[di.at[p]], sems[p], add=True)
        pltpu.async_copy(wz, denS.at[di.at[p]], sems[p], add=True)
    _prep(0, 0)

    def _step(i, p):
        pn = (p + 1) % NBUF
        inxt = jnp.minimum(i + 1, NCH - 1)
        pltpu.make_async_copy(rows4.at[pn], accS.at[di.at[pn]],
                              sems[pn]).wait()
        pltpu.make_async_copy(wz, denS.at[di.at[pn]], sems[pn]).wait()
        _prep(inxt, pn)
        pltpu.make_async_copy(h2_hbm.at[si.at[p]], rows4.at[p],
                              semg[p]).wait()

        def _scale(g, c2):
            w16 = wv[p, pl.ds(g * LANES, LANES)]
            for t in range(LANES):
                w = w16[t]
                for k in range(HC // LANES):
                    sl = pl.ds(k * LANES, LANES)
                    rows4[p, g * LANES + t, sl] = rows4[p, g * LANES + t, sl] * w
            return c2

        lax.fori_loop(0, C // LANES, _scale, 0)
        pltpu.async_copy(rows4.at[p], accS.at[di.at[p]], sems[p], add=True)
        pltpu.async_copy(wv.at[p], denS.at[di.at[p]], sems[p], add=True)

    def _quad(g, carry):
        for p in range(NBUF):
            _step(g * NBUF + p, p)
        return carry

    lax.fori_loop(0, NCH // NBUF, _quad, 0)

    pltpu.make_async_copy(h2_hbm.at[si.at[0]], rows4.at[0], semg[0]).wait()
    for p in range(1, NBUF):
        pltpu.make_async_copy(rows4.at[p], accS.at[di.at[p]], sems[p]).wait()
        pltpu.make_async_copy(wz, denS.at[di.at[p]], sems[p]).wait()
    plsc.subcore_barrier()

    pltpu.sync_copy(accS.at[pl.ds(r0, RPT), :], acc_out.at[cid, pl.ds(r0, RPT), :])
    pltpu.sync_copy(denS.at[pl.ds(r0, RPT)], den_out.at[cid, pl.ds(r0, RPT)])


def _sc_aggregate(src, dst, a, b, h):
    mesh = plsc.VectorSubcoreMesh(core_axis_name="c", subcore_axis_name="s")
    kern = functools.partial(
        pl.kernel,
        mesh=mesh,
        compiler_params=pltpu.CompilerParams(
            needs_layout_passes=False, use_tc_tiling_on_sc=False),
        out_type=[
            jax.ShapeDtypeStruct((2, NP, HC), jnp.float32),
            jax.ShapeDtypeStruct((2, NP), jnp.float32),
        ],
        scratch_types=[
            pltpu.VMEM((NP,), jnp.float32),
            pltpu.VMEM((NP,), jnp.float32),
            pltpu.VMEM((NBUF, C), jnp.int32),
            pltpu.VMEM((NBUF, C), jnp.int32),
            pltpu.VMEM((NBUF, C), jnp.float32),
            pltpu.VMEM((C,), jnp.float32),
            pltpu.VMEM((NBUF, C, HC), jnp.float32),
            pltpu.VMEM_SHARED((NP, HC), jnp.float32),
            pltpu.VMEM_SHARED((NP,), jnp.float32),
        ] + [pltpu.SemaphoreType.DMA] * (2 * NBUF),
    )(_sc_body)
    return kern(src, dst, a, b, h)


_BN_SCALE = 1.0 / math.sqrt(1.0 + 1e-5)


def _epi_body(acc_ref, den_ref, h_ref, ab_ref, bias_ref, gam_ref, bet_ref,
              fcw_ref, fcb_ref, o_ref):
    i = pl.program_id(0)
    den2 = den_ref[:, pl.ds(i * BR, BR)]
    ab = ab_ref[...]
    e = ab[:, 0] + ab[:, 1]
    e = jnp.where(e < 0.0, e * 0.2, e)
    ws = jnp.exp(e)
    acc = jnp.concatenate([acc_ref[0], acc_ref[1]], axis=1)
    acc = acc + ws[:, None] * h_ref[...]
    den = (den2[0] + den2[1]) * 0.5 + ws
    node = acc / (den + 1e-16)[:, None]
    node = node + bias_ref[...]
    node = node * _BN_SCALE * gam_ref[...] + bet_ref[...]
    node = jnp.maximum(node, 0.0)
    o_ref[...] = jnp.dot(node, fcw_ref[...],
                         preferred_element_type=jnp.float32) + fcb_ref[...]


def _epilogue(accp, denp, h, ab, bias, gam, bet, fc_W, fc_b):
    return pl.pallas_call(
        _epi_body,
        grid=(NP // BR,),
        in_specs=[
            pl.BlockSpec((2, BR, HC), lambda i: (0, i, 0)),
            pl.BlockSpec((2, NP), lambda i: (0, 0)),
            pl.BlockSpec((BR, H), lambda i: (i, 0)),
            pl.BlockSpec((BR, 2), lambda i: (i, 0)),
            pl.BlockSpec((1, H), lambda i: (0, 0)),
            pl.BlockSpec((1, H), lambda i: (0, 0)),
            pl.BlockSpec((1, H), lambda i: (0, 0)),
            pl.BlockSpec((H, O), lambda i: (0, 0)),
            pl.BlockSpec((1, O), lambda i: (0, 0)),
        ],
        out_specs=pl.BlockSpec((BR, O), lambda i: (i, 0)),
        out_shape=jax.ShapeDtypeStruct((NP, O), jnp.float32),
    )(accp, denp, h, ab, bias, gam, bet, fc_W, fc_b)


def kernel(x, edge_index, W, att_src, att_dst, bias, bn_gamma, bn_beta,
           fc_W, fc_b):
    x_p = jnp.pad(x, ((0, NP - N0), (0, 0)))
    pad = EP - E0
    pad_idx = N0 + (jnp.arange(pad, dtype=jnp.int32) % (NP - N0))
    src = jnp.concatenate([edge_index[0].astype(jnp.int32), pad_idx])
    dst = jnp.concatenate([edge_index[1].astype(jnp.int32), pad_idx])
    att_mat = jnp.stack([att_src, att_dst], axis=1)

    h, ab = _dense1(x_p, W, att_mat)
    a = ab[:, 0] + 0.0
    b = ab[:, 1] + 0.0
    h2 = h.reshape(2 * NP, HC)
    accp, denp = _sc_aggregate(src, dst, a, b, h2)
    out = _epilogue(accp, denp, h, ab, bias.reshape(1, H),
                    bn_gamma.reshape(1, H), bn_beta.reshape(1, H),
                    fc_W, fc_b.reshape(1, O))
    return out[:N0]


# speedup vs baseline: 41.0538x; 1.3293x over previous
import functools
import math

import jax
import jax.numpy as jnp
from jax import lax
from jax.experimental import pallas as pl
from jax.experimental.pallas import tpu as pltpu
from jax.experimental.pallas import tpu_sc as plsc

N0 = 10000
E0 = 320000
D = 128
H = 128
O = 128

LANES = 16
NP = 10240
C = 128
HC = H // 2
NT = 16
NCH = 160
EP = NCH * NT * C
NBUF = 4
BLK = 8
NBLK = NCH // BLK
QB = 4
RPT = NP // LANES
BR = 256


def _dense_body(x_ref, w_ref, att_ref, h2_ref, ab_ref):
    h = jnp.dot(x_ref[...], w_ref[...], preferred_element_type=jnp.float32)
    h2_ref[0] = h[:, :HC]
    h2_ref[1] = h[:, HC:]
    ab_ref[...] = jnp.dot(h, att_ref[...], preferred_element_type=jnp.float32)


def _dense1(x_p, W, att_mat):
    return pl.pallas_call(
        _dense_body,
        grid=(NP // BR,),
        in_specs=[
            pl.BlockSpec((BR, D), lambda i: (i, 0)),
            pl.BlockSpec((D, H), lambda i: (0, 0)),
            pl.BlockSpec((H, 2), lambda i: (0, 0)),
        ],
        out_specs=[
            pl.BlockSpec((2, BR, HC), lambda i: (0, i, 0)),
            pl.BlockSpec((BR, 2), lambda i: (i, 0)),
        ],
        out_shape=[
            jax.ShapeDtypeStruct((2, NP, HC), jnp.float32),
            jax.ShapeDtypeStruct((NP, 2), jnp.float32),
        ],
    )(x_p, W, att_mat)


def _sc_body(src_hbm, dst_hbm, a_hbm, b_hbm, h3_hbm, acc_out, den_out,
             a_v, b_v, siB, diB, wvB, wz, rows4, accS, denS,
             sg0, sg1, sg2, sg3, ss0, ss1, ss2, ss3, smi):
    cid = lax.axis_index("c")
    sid = lax.axis_index("s")
    semg = [sg0, sg1, sg2, sg3]
    sems = [ss0, ss1, ss2, ss3]
    coff = cid * NP

    pltpu.sync_copy(a_hbm, a_v)
    pltpu.sync_copy(b_hbm, b_v)

    zeros16 = jnp.zeros((LANES,), jnp.float32)

    def _zrow(j, carry):
        for p in range(NBUF):
            for k in range(HC // LANES):
                rows4[p, j, pl.ds(k * LANES, LANES)] = zeros16
        return carry

    lax.fori_loop(0, C, _zrow, 0)
    for k in range(C // LANES):
        wz[pl.ds(k * LANES, LANES)] = zeros16

    r0 = sid * RPT
    for q in range(RPT // C):
        pltpu.sync_copy(rows4.at[0], accS.at[pl.ds(r0 + q * C, C), :])
        pltpu.sync_copy(wz, denS.at[pl.ds(r0 + q * C, C)])
    plsc.subcore_barrier()

    def _wblock(qdst, gsrc):
        def _wrow(j8, carry):
            for j in range(C // LANES):
                sl = pl.ds(j * LANES, LANES)
                s16 = siB[qdst, j8, sl]
                d16 = diB[qdst, j8, sl]
                e = plsc.load_gather(a_v, [s16]) + plsc.load_gather(b_v, [d16])
                e = jnp.where(e < 0.0, e * 0.2, e)
                wvB[qdst, j8, sl] = jnp.exp(e)
                siB[qdst, j8, sl] = s16 + coff
            return carry

        lax.fori_loop(0, BLK, _wrow, 0)

    pltpu.sync_copy(src_hbm.at[sid, 0], siB.at[0])
    pltpu.sync_copy(dst_hbm.at[sid, 0], diB.at[0])
    _wblock(0, 0)
    pltpu.async_copy(src_hbm.at[sid, 1], siB.at[1], smi)
    pltpu.async_copy(dst_hbm.at[sid, 1], diB.at[1], smi)
    for p in range(1, NBUF):
        pltpu.async_copy(rows4.at[p], accS.at[diB.at[0, p]], sems[p], add=True)
        pltpu.async_copy(wz, denS.at[diB.at[0, p]], sems[p], add=True)
    pltpu.async_copy(h3_hbm.at[siB.at[0, 0]], rows4.at[0], semg[0])

    def _block(g, carry):
        q = lax.rem(g, QB)
        qn1 = lax.rem(g + 1, QB)
        qn2 = lax.rem(g + 2, QB)
        gn1 = jnp.minimum(g + 1, NBLK - 1)
        gn2 = jnp.minimum(g + 2, NBLK - 1)
        pltpu.make_async_copy(src_hbm.at[sid, gn1], siB.at[qn1],
                              smi).wait()
        pltpu.make_async_copy(dst_hbm.at[sid, gn1], diB.at[qn1],
                              smi).wait()
        _wblock(qn1, gn1)
        pltpu.async_copy(src_hbm.at[sid, gn2], siB.at[qn2], smi)
        pltpu.async_copy(dst_hbm.at[sid, gn2], diB.at[qn2], smi)

        for j in range(BLK):
            p = j % NBUF
            pn = (p + 1) % NBUF
            qb, jb = (q, j + 1) if j < BLK - 1 else (qn1, 0)
            pltpu.make_async_copy(rows4.at[pn], accS.at[diB.at[q, 0]],
                                  sems[pn]).wait()
            pltpu.make_async_copy(wz, denS.at[diB.at[q, 0]], sems[pn]).wait()
            pltpu.async_copy(h3_hbm.at[siB.at[qb, jb]], rows4.at[pn],
                             semg[pn])
            pltpu.make_async_copy(h3_hbm.at[siB.at[q, j]], rows4.at[p],
                                  semg[p]).wait()

            def _scale(gg, c2, _p=p, _j=j):
                w16 = wvB[q, _j, pl.ds(gg * LANES, LANES)]
                for t in range(LANES):
                    w = w16[t]
                    for k in range(HC // LANES):
                        sl = pl.ds(k * LANES, LANES)
                        rows4[_p, gg * LANES + t, sl] = (
                            rows4[_p, gg * LANES + t, sl] * w)
                return c2

            lax.fori_loop(0, C // LANES, _scale, 0)
            pltpu.async_copy(rows4.at[p], accS.at[diB.at[q, j]], sems[p],
                             add=True)
            pltpu.async_copy(wvB.at[q, j], denS.at[diB.at[q, j]], sems[p],
                             add=True)
        return carry

    lax.fori_loop(0, NBLK, _block, 0)

    qtail = (NBLK + 1) % QB
    pltpu.make_async_copy(src_hbm.at[sid, 0], siB.at[qtail], smi).wait()
    pltpu.make_async_copy(dst_hbm.at[sid, 0], diB.at[qtail], smi).wait()
    pltpu.make_async_copy(h3_hbm.at[siB.at[0, 0]], rows4.at[0], semg[0]).wait()
    for p in range(1, NBUF):
        pltpu.make_async_copy(rows4.at[p], accS.at[diB.at[0, 0]],
                              sems[p]).wait()
        pltpu.make_async_copy(wz, denS.at[diB.at[0, 0]], sems[p]).wait()
    plsc.subcore_barrier()

    pltpu.sync_copy(accS.at[pl.ds(r0, RPT), :], acc_out.at[cid, pl.ds(r0, RPT), :])
    pltpu.sync_copy(denS.at[pl.ds(r0, RPT)], den_out.at[cid, pl.ds(r0, RPT)])


def _sc_aggregate(src, dst, a, b, h):
    mesh = plsc.VectorSubcoreMesh(core_axis_name="c", subcore_axis_name="s")
    kern = functools.partial(
        pl.kernel,
        mesh=mesh,
        compiler_params=pltpu.CompilerParams(
            needs_layout_passes=False, use_tc_tiling_on_sc=False),
        out_type=[
            jax.ShapeDtypeStruct((2, NP, HC), jnp.float32),
            jax.ShapeDtypeStruct((2, NP), jnp.float32),
        ],
        scratch_types=[
            pltpu.VMEM((NP,), jnp.float32),
            pltpu.VMEM((NP,), jnp.float32),
            pltpu.VMEM((QB, BLK, C), jnp.int32),
            pltpu.VMEM((QB, BLK, C), jnp.int32),
            pltpu.VMEM((QB, BLK, C), jnp.float32),
            pltpu.VMEM((C,), jnp.float32),
            pltpu.VMEM((NBUF, C, HC), jnp.float32),
            pltpu.VMEM_SHARED((NP, HC), jnp.float32),
            pltpu.VMEM_SHARED((NP,), jnp.float32),
        ] + [pltpu.SemaphoreType.DMA] * (2 * NBUF + 1),
    )(_sc_body)
    return kern(src, dst, a, b, h)


_BN_SCALE = 1.0 / math.sqrt(1.0 + 1e-5)


def _epi_body(acc_ref, den_ref, h_ref, ab_ref, bias_ref, gam_ref, bet_ref,
              fcw_ref, fcb_ref, o_ref):
    i = pl.program_id(0)
    den2 = den_ref[:, pl.ds(i * BR, BR)]
    ab = ab_ref[...]
    e = ab[:, 0] + ab[:, 1]
    e = jnp.where(e < 0.0, e * 0.2, e)
    ws = jnp.exp(e)
    acc = jnp.concatenate([acc_ref[0], acc_ref[1]], axis=1)
    hh = jnp.concatenate([h_ref[0], h_ref[1]], axis=1)
    acc = acc + ws[:, None] * hh
    den = (den2[0] + den2[1]) * 0.5 + ws
    node = acc / (den + 1e-16)[:, None]
    node = node + bias_ref[...]
    node = node * _BN_SCALE * gam_ref[...] + bet_ref[...]
    node = jnp.maximum(node, 0.0)
    o_ref[...] = jnp.dot(node, fcw_ref[...],
                         preferred_element_type=jnp.float32) + fcb_ref[...]


def _epilogue(accp, denp, h, ab, bias, gam, bet, fc_W, fc_b):
    return pl.pallas_call(
        _epi_body,
        grid=(NP // BR,),
        in_specs=[
            pl.BlockSpec((2, BR, HC), lambda i: (0, i, 0)),
            pl.BlockSpec((2, NP), lambda i: (0, 0)),
            pl.BlockSpec((2, BR, HC), lambda i: (0, i, 0)),
            pl.BlockSpec((BR, 2), lambda i: (i, 0)),
            pl.BlockSpec((1, H), lambda i: (0, 0)),
            pl.BlockSpec((1, H), lambda i: (0, 0)),
            pl.BlockSpec((1, H), lambda i: (0, 0)),
            pl.BlockSpec((H, O), lambda i: (0, 0)),
            pl.BlockSpec((1, O), lambda i: (0, 0)),
        ],
        out_specs=pl.BlockSpec((BR, O), lambda i: (i, 0)),
        out_shape=jax.ShapeDtypeStruct((NP, O), jnp.float32),
    )(accp, denp, h, ab, bias, gam, bet, fc_W, fc_b)


def kernel(x, edge_index, W, att_src, att_dst, bias, bn_gamma, bn_beta,
           fc_W, fc_b):
    x_p = jnp.pad(x, ((0, NP - N0), (0, 0)))
    pad = EP - E0
    pad_idx = N0 + (jnp.arange(pad, dtype=jnp.int32) % (NP - N0))
    src = jnp.concatenate([edge_index[0].astype(jnp.int32), pad_idx]
                          ).reshape(NT, NBLK, BLK, C)
    dst = jnp.concatenate([edge_index[1].astype(jnp.int32), pad_idx]
                          ).reshape(NT, NBLK, BLK, C)
    att_mat = jnp.stack([att_src, att_dst], axis=1)

    h2, ab = _dense1(x_p, W, att_mat)
    a = ab[:, 0] + 0.0
    b = ab[:, 1] + 0.0
    h3 = h2.reshape(2 * NP, HC)
    accp, denp = _sc_aggregate(src, dst, a, b, h3)
    out = _epilogue(accp, denp, h2, ab, bias.reshape(1, H),
                    bn_gamma.reshape(1, H), bn_beta.reshape(1, H),
                    fc_W, fc_b.reshape(1, O))
    return out[:N0]


# speedup vs baseline: 45.5503x; 1.1095x over previous
import functools
import math

import jax
import jax.numpy as jnp
from jax import lax
from jax.experimental import pallas as pl
from jax.experimental.pallas import tpu as pltpu
from jax.experimental.pallas import tpu_sc as plsc

N0 = 10000
E0 = 320000
D = 128
H = 128
O = 128

LANES = 16
NP = 10240
C = 128
HC = H // 2
NT = 16
NCH = 160
EP = NCH * NT * C
NBUF = 4
BLK = 8
NBLK = NCH // BLK
QB = 4
RPT = NP // LANES
BR = 256


def _dense_body(x_ref, w_ref, att_ref, h2_ref, ab_ref):
    h = jnp.dot(x_ref[...], w_ref[...], preferred_element_type=jnp.float32)
    h2_ref[0] = h[:, :HC]
    h2_ref[1] = h[:, HC:]
    ab_ref[...] = jnp.dot(h, att_ref[...], preferred_element_type=jnp.float32)


def _dense1(x_p, W, att_mat):
    return pl.pallas_call(
        _dense_body,
        grid=(NP // BR,),
        in_specs=[
            pl.BlockSpec((BR, D), lambda i: (i, 0)),
            pl.BlockSpec((D, H), lambda i: (0, 0)),
            pl.BlockSpec((H, 2), lambda i: (0, 0)),
        ],
        out_specs=[
            pl.BlockSpec((2, BR, HC), lambda i: (0, i, 0)),
            pl.BlockSpec((BR, 2), lambda i: (i, 0)),
        ],
        out_shape=[
            jax.ShapeDtypeStruct((2, NP, HC), jnp.float32),
            jax.ShapeDtypeStruct((NP, 2), jnp.float32),
        ],
    )(x_p, W, att_mat)


def _sc_body(src_hbm, dst_hbm, ab_hbm, h3_hbm, acc_out, den_out,
             ab_v, siB, diB, wvB, wz, rows4, accS, denS,
             sg0, sg1, sg2, sg3, ss0, ss1, ss2, ss3, smi):
    cid = lax.axis_index("c")
    sid = lax.axis_index("s")
    semg = [sg0, sg1, sg2, sg3]
    sems = [ss0, ss1, ss2, ss3]
    coff = cid * NP

    pltpu.sync_copy(ab_hbm, ab_v)

    zeros16 = jnp.zeros((LANES,), jnp.float32)

    def _zrow(j, carry):
        for p in range(NBUF):
            for k in range(HC // LANES):
                rows4[p, j, pl.ds(k * LANES, LANES)] = zeros16
        return carry

    lax.fori_loop(0, C, _zrow, 0)
    for k in range(C // LANES):
        wz[pl.ds(k * LANES, LANES)] = zeros16

    r0 = sid * RPT
    for q in range(RPT // C):
        pltpu.sync_copy(rows4.at[0], accS.at[pl.ds(r0 + q * C, C), :])
        pltpu.sync_copy(wz, denS.at[pl.ds(r0 + q * C, C)])
    plsc.subcore_barrier()

    def _wblock(qdst, gsrc):
        @plsc.parallel_loop(0, BLK, 1)
        def _wrow(j8):
            for j in range(C // LANES):
                sl = pl.ds(j * LANES, LANES)
                s16 = siB[qdst, j8, sl]
                d16 = diB[qdst, j8, sl]
                e = (plsc.load_gather(ab_v, [s16 * 2])
                     + plsc.load_gather(ab_v, [d16 * 2 + 1]))
                e = jnp.where(e < 0.0, e * 0.2, e)
                wvB[qdst, j8, sl] = jnp.exp(e)
                siB[qdst, j8, sl] = s16 + coff

    pltpu.sync_copy(src_hbm.at[sid, 0], siB.at[0])
    pltpu.sync_copy(dst_hbm.at[sid, 0], diB.at[0])
    _wblock(0, 0)
    pltpu.async_copy(src_hbm.at[sid, 1], siB.at[1], smi)
    pltpu.async_copy(dst_hbm.at[sid, 1], diB.at[1], smi)
    for p in range(1, NBUF):
        pltpu.async_copy(rows4.at[p], accS.at[diB.at[0, p]], sems[p], add=True)
        pltpu.async_copy(wz, denS.at[diB.at[0, p]], sems[p], add=True)
    pltpu.async_copy(h3_hbm.at[siB.at[0, 0]], rows4.at[0], semg[0])

    def _block(g, carry):
        q = lax.rem(g, QB)
        qn1 = lax.rem(g + 1, QB)
        qn2 = lax.rem(g + 2, QB)
        gn1 = jnp.minimum(g + 1, NBLK - 1)
        gn2 = jnp.minimum(g + 2, NBLK - 1)
        pltpu.make_async_copy(src_hbm.at[sid, gn1], siB.at[qn1],
                              smi).wait()
        pltpu.make_async_copy(dst_hbm.at[sid, gn1], diB.at[qn1],
                              smi).wait()
        _wblock(qn1, gn1)
        pltpu.async_copy(src_hbm.at[sid, gn2], siB.at[qn2], smi)
        pltpu.async_copy(dst_hbm.at[sid, gn2], diB.at[qn2], smi)

        for j in range(BLK):
            p = j % NBUF
            pn = (p + 1) % NBUF
            qb, jb = (q, j + 1) if j < BLK - 1 else (qn1, 0)
            pltpu.make_async_copy(rows4.at[pn], accS.at[diB.at[q, 0]],
                                  sems[pn]).wait()
            pltpu.make_async_copy(wz, denS.at[diB.at[q, 0]], sems[pn]).wait()
            pltpu.async_copy(h3_hbm.at[siB.at[qb, jb]], rows4.at[pn],
                             semg[pn])
            pltpu.make_async_copy(h3_hbm.at[siB.at[q, j]], rows4.at[p],
                                  semg[p]).wait()

            @plsc.parallel_loop(0, C // LANES, 1, unroll=2)
            def _scale(gg, _p=p, _j=j):
                w16 = wvB[q, _j, pl.ds(gg * LANES, LANES)]
                for t in range(LANES):
                    w = w16[t]
                    for k in range(HC // LANES):
                        sl = pl.ds(k * LANES, LANES)
                        rows4[_p, gg * LANES + t, sl] = (
                            rows4[_p, gg * LANES + t, sl] * w)
            pltpu.async_copy(rows4.at[p], accS.at[diB.at[q, j]], sems[p],
                             add=True)
            pltpu.async_copy(wvB.at[q, j], denS.at[diB.at[q, j]], sems[p],
                             add=True)
        return carry

    lax.fori_loop(0, NBLK, _block, 0)

    qtail = (NBLK + 1) % QB
    pltpu.make_async_copy(src_hbm.at[sid, 0], siB.at[qtail], smi).wait()
    pltpu.make_async_copy(dst_hbm.at[sid, 0], diB.at[qtail], smi).wait()
    pltpu.make_async_copy(h3_hbm.at[siB.at[0, 0]], rows4.at[0], semg[0]).wait()
    for p in range(1, NBUF):
        pltpu.make_async_copy(rows4.at[p], accS.at[diB.at[0, 0]],
                              sems[p]).wait()
        pltpu.make_async_copy(wz, denS.at[diB.at[0, 0]], sems[p]).wait()
    plsc.subcore_barrier()

    pltpu.sync_copy(accS.at[pl.ds(r0, RPT), :], acc_out.at[cid, pl.ds(r0, RPT), :])
    pltpu.sync_copy(denS.at[pl.ds(r0, RPT)], den_out.at[cid, pl.ds(r0, RPT)])


def _sc_aggregate(src, dst, ab_flat, h):
    mesh = plsc.VectorSubcoreMesh(core_axis_name="c", subcore_axis_name="s")
    kern = functools.partial(
        pl.kernel,
        mesh=mesh,
        compiler_params=pltpu.CompilerParams(
            needs_layout_passes=False, use_tc_tiling_on_sc=False),
        out_type=[
            jax.ShapeDtypeStruct((2, NP, HC), jnp.float32),
            jax.ShapeDtypeStruct((2, NP), jnp.float32),
        ],
        scratch_types=[
            pltpu.VMEM((2 * NP,), jnp.float32),
            pltpu.VMEM((QB, BLK, C), jnp.int32),
            pltpu.VMEM((QB, BLK, C), jnp.int32),
            pltpu.VMEM((QB, BLK, C), jnp.float32),
            pltpu.VMEM((C,), jnp.float32),
            pltpu.VMEM((NBUF, C, HC), jnp.float32),
            pltpu.VMEM_SHARED((NP, HC), jnp.float32),
            pltpu.VMEM_SHARED((NP,), jnp.float32),
        ] + [pltpu.SemaphoreType.DMA] * (2 * NBUF + 1),
    )(_sc_body)
    return kern(src, dst, ab_flat, h)


_BN_SCALE = 1.0 / math.sqrt(1.0 + 1e-5)


def _epi_body(acc_ref, den_ref, h_ref, ab_ref, bias_ref, gam_ref, bet_ref,
              fcw_ref, fcb_ref, o_ref):
    i = pl.program_id(0)
    den2 = den_ref[:, pl.ds(i * BR, BR)]
    ab = ab_ref[...]
    e = ab[:, 0] + ab[:, 1]
    e = jnp.where(e < 0.0, e * 0.2, e)
    ws = jnp.exp(e)
    acc = jnp.concatenate([acc_ref[0], acc_ref[1]], axis=1)
    hh = jnp.concatenate([h_ref[0], h_ref[1]], axis=1)
    acc = acc + ws[:, None] * hh
    den = (den2[0] + den2[1]) * 0.5 + ws
    node = acc / (den + 1e-16)[:, None]
    node = node + bias_ref[...]
    node = node * _BN_SCALE * gam_ref[...] + bet_ref[...]
    node = jnp.maximum(node, 0.0)
    o_ref[...] = jnp.dot(node, fcw_ref[...],
                         preferred_element_type=jnp.float32) + fcb_ref[...]


def _epilogue(accp, denp, h, ab, bias, gam, bet, fc_W, fc_b):
    return pl.pallas_call(
        _epi_body,
        grid=(NP // BR,),
        in_specs=[
            pl.BlockSpec((2, BR, HC), lambda i: (0, i, 0)),
            pl.BlockSpec((2, NP), lambda i: (0, 0)),
            pl.BlockSpec((2, BR, HC), lambda i: (0, i, 0)),
            pl.BlockSpec((BR, 2), lambda i: (i, 0)),
            pl.BlockSpec((1, H), lambda i: (0, 0)),
            pl.BlockSpec((1, H), lambda i: (0, 0)),
            pl.BlockSpec((1, H), lambda i: (0, 0)),
            pl.BlockSpec((H, O), lambda i: (0, 0)),
            pl.BlockSpec((1, O), lambda i: (0, 0)),
        ],
        out_specs=pl.BlockSpec((BR, O), lambda i: (i, 0)),
        out_shape=jax.ShapeDtypeStruct((N0, O), jnp.float32),
    )(accp, denp, h, ab, bias, gam, bet, fc_W, fc_b)


def kernel(x, edge_index, W, att_src, att_dst, bias, bn_gamma, bn_beta,
           fc_W, fc_b):
    pad = EP - E0
    pad_idx = N0 + (jnp.arange(pad, dtype=jnp.int32) % (NP - N0))
    src = jnp.concatenate([edge_index[0].astype(jnp.int32), pad_idx]
                          ).reshape(NT, NBLK, BLK, C)
    dst = jnp.concatenate([edge_index[1].astype(jnp.int32), pad_idx]
                          ).reshape(NT, NBLK, BLK, C)
    att_mat = jnp.stack([att_src, att_dst], axis=1)

    h2, ab = _dense1(x, W, att_mat)
    ab_flat = ab.reshape(2 * NP)
    h3 = h2.reshape(2 * NP, HC)
    accp, denp = _sc_aggregate(src, dst, ab_flat, h3)
    return _epilogue(accp, denp, h2, ab, bias.reshape(1, H),
                     bn_gamma.reshape(1, H), bn_beta.reshape(1, H),
                     fc_W, fc_b.reshape(1, O))


# speedup vs baseline: 48.3665x; 1.0618x over previous
import functools
import math

import jax
import jax.numpy as jnp
from jax import lax
from jax.experimental import pallas as pl
from jax.experimental.pallas import tpu as pltpu
from jax.experimental.pallas import tpu_sc as plsc

N0 = 10000
E0 = 320000
D = 128
H = 128
O = 128

LANES = 16
NP = 10240
C = 128
HC = H // 2
NT = 16
NCH = 160
EP = NCH * NT * C
NBUF = 4
BLK = 8
NBLK = NCH // BLK
QB = 4
RPT = NP // LANES
BR = 256


def _dense_body(x_ref, w_ref, att_ref, h2_ref, ab_ref):
    h = jnp.dot(x_ref[...], w_ref[...], preferred_element_type=jnp.float32)
    h2_ref[0] = h[:, :HC]
    h2_ref[1] = h[:, HC:]
    ab_ref[...] = jnp.dot(h, att_ref[...], preferred_element_type=jnp.float32)


def _dense1(x_p, W, att_mat):
    return pl.pallas_call(
        _dense_body,
        grid=(NP // BR,),
        in_specs=[
            pl.BlockSpec((BR, D), lambda i: (i, 0)),
            pl.BlockSpec((D, H), lambda i: (0, 0)),
            pl.BlockSpec((H, 2), lambda i: (0, 0)),
        ],
        out_specs=[
            pl.BlockSpec((2, BR, HC), lambda i: (0, i, 0)),
            pl.BlockSpec((BR, 2), lambda i: (i, 0)),
        ],
        out_shape=[
            jax.ShapeDtypeStruct((2, NP, HC), jnp.float32),
            jax.ShapeDtypeStruct((NP, 2), jnp.float32),
        ],
    )(x_p, W, att_mat)


def _sc_body(src_hbm, dst_hbm, ab_hbm, h3_hbm, acc_out, den_out,
             ab_v, siB, diB, wvB, wz, rows4, accS, denS,
             sg0, sg1, sg2, sg3, ss0, ss1, ss2, ss3, smi):
    cid = lax.axis_index("c")
    sid = lax.axis_index("s")
    semg = [sg0, sg1, sg2, sg3]
    sems = [ss0, ss1, ss2, ss3]
    coff = cid * NP

    pltpu.sync_copy(ab_hbm, ab_v)

    zeros16 = jnp.zeros((LANES,), jnp.float32)

    def _zrow(j, carry):
        for p in range(NBUF):
            for k in range(HC // LANES):
                rows4[p, j, pl.ds(k * LANES, LANES)] = zeros16
        return carry

    lax.fori_loop(0, C, _zrow, 0)
    for k in range(C // LANES):
        wz[pl.ds(k * LANES, LANES)] = zeros16

    r0 = sid * RPT
    for q in range(RPT // C):
        pltpu.sync_copy(rows4.at[0], accS.at[pl.ds(r0 + q * C, C), :])
        pltpu.sync_copy(wz, denS.at[pl.ds(r0 + q * C, C)])
    plsc.subcore_barrier()

    def _wblock(qdst, gsrc):
        @plsc.parallel_loop(0, BLK, 1)
        def _wrow(j8):
            for j in range(C // LANES):
                sl = pl.ds(j * LANES, LANES)
                s16 = siB[qdst, j8, sl]
                d16 = diB[qdst, j8, sl]
                e = (plsc.load_gather(ab_v, [s16 * 2])
                     + plsc.load_gather(ab_v, [d16 * 2 + 1]))
                e = jnp.where(e < 0.0, e * 0.2, e)
                wvB[qdst, j8, sl] = jnp.exp(e)
                siB[qdst, j8, sl] = s16 + coff

    pltpu.sync_copy(src_hbm.at[sid, 0], siB.at[0])
    pltpu.sync_copy(dst_hbm.at[sid, 0], diB.at[0])
    _wblock(0, 0)
    pltpu.async_copy(src_hbm.at[sid, 1], siB.at[1], smi)
    pltpu.async_copy(dst_hbm.at[sid, 1], diB.at[1], smi)
    for p in range(2, NBUF):
        pltpu.async_copy(rows4.at[p], accS.at[diB.at[0, p]], sems[p], add=True)
        pltpu.async_copy(wz, denS.at[diB.at[0, p]], sems[p], add=True)
    pltpu.async_copy(h3_hbm.at[siB.at[0, 0]], rows4.at[0], semg[0])
    pltpu.async_copy(h3_hbm.at[siB.at[0, 1]], rows4.at[1], semg[1])

    def _block(g, carry):
        q = lax.rem(g, QB)
        qn1 = lax.rem(g + 1, QB)
        qn2 = lax.rem(g + 2, QB)
        gn1 = jnp.minimum(g + 1, NBLK - 1)
        gn2 = jnp.minimum(g + 2, NBLK - 1)
        pltpu.make_async_copy(src_hbm.at[sid, gn1], siB.at[qn1],
                              smi).wait()
        pltpu.make_async_copy(dst_hbm.at[sid, gn1], diB.at[qn1],
                              smi).wait()
        _wblock(qn1, gn1)
        pltpu.async_copy(src_hbm.at[sid, gn2], siB.at[qn2], smi)
        pltpu.async_copy(dst_hbm.at[sid, gn2], diB.at[qn2], smi)

        for j in range(BLK):
            p = j % NBUF
            pn = (p + 2) % NBUF
            qb, jb = (q, j + 2) if j < BLK - 2 else (qn1, j - (BLK - 2))
            pltpu.make_async_copy(rows4.at[pn], accS.at[diB.at[q, 0]],
                                  sems[pn]).wait()
            pltpu.make_async_copy(wz, denS.at[diB.at[q, 0]], sems[pn]).wait()
            pltpu.async_copy(h3_hbm.at[siB.at[qb, jb]], rows4.at[pn],
                             semg[pn])
            pltpu.make_async_copy(h3_hbm.at[siB.at[q, j]], rows4.at[p],
                                  semg[p]).wait()

            @plsc.parallel_loop(0, C // LANES, 1, unroll=2)
            def _scale(gg, _p=p, _j=j):
                w16 = wvB[q, _j, pl.ds(gg * LANES, LANES)]
                for t in range(LANES):
                    w = w16[t]
                    for k in range(HC // LANES):
                        sl = pl.ds(k * LANES, LANES)
                        rows4[_p, gg * LANES + t, sl] = (
                            rows4[_p, gg * LANES + t, sl] * w)
            pltpu.async_copy(rows4.at[p], accS.at[diB.at[q, j]], sems[p],
                             add=True)
            pltpu.async_copy(wvB.at[q, j], denS.at[diB.at[q, j]], sems[p],
                             add=True)
        return carry

    lax.fori_loop(0, NBLK, _block, 0)

    qtail = (NBLK + 1) % QB
    pltpu.make_async_copy(src_hbm.at[sid, 0], siB.at[qtail], smi).wait()
    pltpu.make_async_copy(dst_hbm.at[sid, 0], diB.at[qtail], smi).wait()
    pltpu.make_async_copy(h3_hbm.at[siB.at[0, 0]], rows4.at[0], semg[0]).wait()
    pltpu.make_async_copy(h3_hbm.at[siB.at[0, 0]], rows4.at[1], semg[1]).wait()
    for p in range(2, NBUF):
        pltpu.make_async_copy(rows4.at[p], accS.at[diB.at[0, 0]],
                              sems[p]).wait()
        pltpu.make_async_copy(wz, denS.at[diB.at[0, 0]], sems[p]).wait()
    plsc.subcore_barrier()

    pltpu.sync_copy(accS.at[pl.ds(r0, RPT), :], acc_out.at[cid, pl.ds(r0, RPT), :])
    pltpu.sync_copy(denS.at[pl.ds(r0, RPT)], den_out.at[cid, pl.ds(r0, RPT)])


def _sc_aggregate(src, dst, ab_flat, h):
    mesh = plsc.VectorSubcoreMesh(core_axis_name="c", subcore_axis_name="s")
    kern = functools.partial(
        pl.kernel,
        mesh=mesh,
        compiler_params=pltpu.CompilerParams(
            needs_layout_passes=False, use_tc_tiling_on_sc=False),
        out_type=[
            jax.ShapeDtypeStruct((2, NP, HC), jnp.float32),
            jax.ShapeDtypeStruct((2, NP), jnp.float32),
        ],
        scratch_types=[
            pltpu.VMEM((2 * NP,), jnp.float32),
            pltpu.VMEM((QB, BLK, C), jnp.int32),
            pltpu.VMEM((QB, BLK, C), jnp.int32),
            pltpu.VMEM((QB, BLK, C), jnp.float32),
            pltpu.VMEM((C,), jnp.float32),
            pltpu.VMEM((NBUF, C, HC), jnp.float32),
            pltpu.VMEM_SHARED((NP, HC), jnp.float32),
            pltpu.VMEM_SHARED((NP,), jnp.float32),
        ] + [pltpu.SemaphoreType.DMA] * (2 * NBUF + 1),
    )(_sc_body)
    return kern(src, dst, ab_flat, h)


_BN_SCALE = 1.0 / math.sqrt(1.0 + 1e-5)


def _epi_body(acc_ref, den_ref, h_ref, ab_ref, bias_ref, gam_ref, bet_ref,
              fcw_ref, fcb_ref, o_ref):
    i = pl.program_id(0)
    den2 = den_ref[:, pl.ds(i * BR, BR)]
    ab = ab_ref[...]
    e = ab[:, 0] + ab[:, 1]
    e = jnp.where(e < 0.0, e * 0.2, e)
    ws = jnp.exp(e)
    acc = jnp.concatenate([acc_ref[0], acc_ref[1]], axis=1)
    hh = jnp.concatenate([h_ref[0], h_ref[1]], axis=1)
    acc = acc + ws[:, None] * hh
    den = (den2[0] + den2[1]) * 0.5 + ws
    node = acc / (den + 1e-16)[:, None]
    node = node + bias_ref[...]
    node = node * _BN_SCALE * gam_ref[...] + bet_ref[...]
    node = jnp.maximum(node, 0.0)
    o_ref[...] = jnp.dot(node, fcw_ref[...],
                         preferred_element_type=jnp.float32) + fcb_ref[...]


def _epilogue(accp, denp, h, ab, bias, gam, bet, fc_W, fc_b):
    return pl.pallas_call(
        _epi_body,
        grid=(NP // BR,),
        in_specs=[
            pl.BlockSpec((2, BR, HC), lambda i: (0, i, 0)),
            pl.BlockSpec((2, NP), lambda i: (0, 0)),
            pl.BlockSpec((2, BR, HC), lambda i: (0, i, 0)),
            pl.BlockSpec((BR, 2), lambda i: (i, 0)),
            pl.BlockSpec((1, H), lambda i: (0, 0)),
            pl.BlockSpec((1, H), lambda i: (0, 0)),
            pl.BlockSpec((1, H), lambda i: (0, 0)),
            pl.BlockSpec((H, O), lambda i: (0, 0)),
            pl.BlockSpec((1, O), lambda i: (0, 0)),
        ],
        out_specs=pl.BlockSpec((BR, O), lambda i: (i, 0)),
        out_shape=jax.ShapeDtypeStruct((N0, O), jnp.float32),
    )(accp, denp, h, ab, bias, gam, bet, fc_W, fc_b)


def kernel(x, edge_index, W, att_src, att_dst, bias, bn_gamma, bn_beta,
           fc_W, fc_b):
    pad = EP - E0
    pad_idx = N0 + (jnp.arange(pad, dtype=jnp.int32) % (NP - N0))
    src = jnp.concatenate([edge_index[0].astype(jnp.int32), pad_idx]
                          ).reshape(NT, NBLK, BLK, C)
    dst = jnp.concatenate([edge_index[1].astype(jnp.int32), pad_idx]
                          ).reshape(NT, NBLK, BLK, C)
    att_mat = jnp.stack([att_src, att_dst], axis=1)

    h2, ab = _dense1(x, W, att_mat)
    ab_flat = ab.reshape(2 * NP)
    h3 = h2.reshape(2 * NP, HC)
    accp, denp = _sc_aggregate(src, dst, ab_flat, h3)
    return _epilogue(accp, denp, h2, ab, bias.reshape(1, H),
                     bn_gamma.reshape(1, H), bn_beta.reshape(1, H),
                     fc_W, fc_b.reshape(1, O))
